# Initial kernel scaffold; baseline (speedup 1.0000x reference)
import jax
import jax.numpy as jnp
from jax.experimental import pallas as pl


def kernel(yhat, target):
    raise NotImplementedError("write your pallas kernel here")


# speedup vs baseline: 9.0959x; 9.0959x over previous
import math

import jax
import jax.numpy as jnp
from jax import lax
from jax.experimental import pallas as pl
from jax.experimental.pallas import tpu as pltpu

V = 32768
N = 4096
PAD = 0
EPS = 0.1 / (V - 2)
COEF = 1.0 - 0.1 - EPS
CONST = 0.9 * math.log(0.9) + 0.1 * math.log(EPS)

R = 256
KBLK = 8192
NR = N // R
NK = V // KBLK


def _body(yhat_ref, tgt_ref, out_ref, acc_ref):
    r = pl.program_id(0)
    k = pl.program_id(1)
    blk = yhat_ref[...]
    tgt = tgt_ref[0]
    colid = k * KBLK + lax.broadcasted_iota(jnp.int32, blk.shape, 1)
    full = jnp.sum(blk, axis=1, keepdims=True)
    pick = jnp.sum(jnp.where(colid == tgt, blk, 0.0), axis=1, keepdims=True)
    q = EPS * full + COEF * pick

    @pl.when(k == 0)
    def _():
        acc_ref[...] = q - EPS * yhat_ref[:, 0:1]

    @pl.when(k > 0)
    def _():
        acc_ref[...] += q

    @pl.when(k == NK - 1)
    def _():
        mask = tgt != PAD
        total = jnp.sum(jnp.where(mask, CONST - acc_ref[...], 0.0))
        total = total.reshape(1, 1)

        @pl.when(r == 0)
        def _():
            out_ref[...] = total

        @pl.when(r > 0)
        def _():
            out_ref[...] += total


def kernel(yhat, target):
    tgt3 = target.reshape(NR, R, 1)
    out = pl.pallas_call(
        _body,
        grid=(NR, NK),
        in_specs=[
            pl.BlockSpec((R, KBLK), lambda r, k: (r, k)),
            pl.BlockSpec((1, R, 1), lambda r, k: (r, 0, 0)),
        ],
        out_specs=pl.BlockSpec((1, 1), lambda r, k: (0, 0)),
        out_shape=jax.ShapeDtypeStruct((1, 1), jnp.float32),
        scratch_shapes=[pltpu.VMEM((R, 1), jnp.float32)],
        compiler_params=pltpu.CompilerParams(
            dimension_semantics=("arbitrary", "arbitrary")),
    )(yhat, tgt3)
    return out[0, 0]
